# baseline (speedup 1.0000x reference)
import jax
import jax.numpy as jnp
from jax.experimental import pallas as pl
from jax.experimental.pallas import tpu as pltpu

_VOCAB = 1000
_ROWS = 4096
_COLS = 20
_BLK = 64
_NBUF = 4
_GRID = _ROWS // _BLK


def _onehot_block(x_ref, o_hbm, *scratch):
    bufs = scratch[:_NBUF]
    sems = scratch[_NBUF]
    i = pl.program_id(0)
    slot = jax.lax.rem(i, _NBUF)
    ids = jax.lax.broadcasted_iota(jnp.int32, (_BLK, _COLS, _VOCAB), 2)
    vals = (ids == x_ref[...][:, :, None]).astype(jnp.float32)
    for k in range(_NBUF):
        @pl.when(slot == k)
        def _():
            copy = pltpu.make_async_copy(
                bufs[k], o_hbm.at[pl.ds(i * _BLK, _BLK)], sems.at[k])
            @pl.when(i >= _NBUF)
            def _():
                copy.wait()
            bufs[k][...] = vals
            copy.start()

    @pl.when(i == _GRID - 1)
    def _():
        for k in range(_NBUF):
            pltpu.make_async_copy(
                bufs[k], o_hbm.at[pl.ds(0, _BLK)], sems.at[k]).wait()


def kernel(x):
    return pl.pallas_call(
        _onehot_block,
        grid=(_GRID,),
        in_specs=[pl.BlockSpec((_BLK, _COLS), lambda i: (i, 0))],
        out_specs=pl.BlockSpec(memory_space=pl.ANY),
        out_shape=jax.ShapeDtypeStruct((_ROWS, _COLS, _VOCAB), jnp.float32),
        scratch_shapes=(
            [pltpu.VMEM((_BLK, _COLS, _VOCAB), jnp.float32)] * _NBUF
            + [pltpu.SemaphoreType.DMA((_NBUF,))]
        ),
    )(x)


# speedup vs baseline: 1.0132x; 1.0132x over previous
import jax
import jax.numpy as jnp
from jax.experimental import pallas as pl
from jax.experimental.pallas import tpu as pltpu

_VOCAB = 1000
_ROWS = 4096
_COLS = 20
_BLK = 64
_NBUF = 4
_GRID = _ROWS // _BLK


def _onehot_block(x_ref, o_hbm, *scratch):
    bufs = scratch[:_NBUF]
    sems = scratch[_NBUF:]
    i = pl.program_id(0)
    slot = jax.lax.rem(i, _NBUF)
    ids = jax.lax.broadcasted_iota(jnp.int32, (_BLK, _COLS, _VOCAB), 2)
    vals = (ids == x_ref[...][:, :, None]).astype(jnp.float32)
    for k in range(_NBUF):
        @pl.when(slot == k)
        def _():
            copy = pltpu.make_async_copy(
                bufs[k], o_hbm.at[pl.ds(i * _BLK, _BLK)], sems[k])
            @pl.when(i >= _NBUF)
            def _():
                copy.wait()
            bufs[k][...] = vals
            copy.start()

    @pl.when(i == _GRID - 1)
    def _():
        for k in range(_NBUF):
            pltpu.make_async_copy(
                bufs[k], o_hbm.at[pl.ds(0, _BLK)], sems[k]).wait()


def kernel(x):
    return pl.pallas_call(
        _onehot_block,
        grid=(_GRID,),
        in_specs=[pl.BlockSpec((_BLK, _COLS), lambda i: (i, 0))],
        out_specs=pl.BlockSpec(memory_space=pl.ANY),
        out_shape=jax.ShapeDtypeStruct((_ROWS, _COLS, _VOCAB), jnp.float32),
        scratch_shapes=(
            [pltpu.VMEM((_BLK, _COLS, _VOCAB), jnp.float32)] * _NBUF
            + [pltpu.SemaphoreType.DMA] * _NBUF
        ),
    )(x)
